# baseline (speedup 1.0000x reference)
import jax
import jax.numpy as jnp
from jax import lax
from jax.experimental import pallas as pl
from jax.experimental.pallas import tpu as pltpu
from jax.experimental.pallas import tpu_sc as plsc

M = 400000
C = 128
R = 128
NUM_CHUNKS = M // R
NC = 2
NS = 16
NW = NC * NS
ITERS = -(-NUM_CHUNKS // NW)
NSTAGE = ITERS // 2
NBUF = 3


def _sc_body(x_hbm, up_hbm, idxp_hbm, out_hbm, idx_v, up_v, lsem, gsem, ssem):
    wid = lax.axis_index("s") * NC + lax.axis_index("c")
    c0 = ITERS * wid
    ni = jnp.minimum(ITERS, NUM_CHUNKS - c0)
    nsf = ni // 2
    tail = ni - 2 * nsf

    pltpu.sync_copy(idxp_hbm.at[wid], idx_v)

    def ldesc(s, b):
        return pltpu.make_async_copy(
            up_hbm.at[pl.ds((c0 + 2 * s) * R, 2 * R)], up_v.at[b], lsem.at[b])

    def gdesc(t, s, b):
        return pltpu.make_async_copy(
            x_hbm.at[idx_v.at[2 * s + t]],
            up_v.at[b, pl.ds(t * R, R)], gsem.at[b])

    def sdesc(s, b):
        return pltpu.make_async_copy(
            up_v.at[b], out_hbm.at[pl.ds((c0 + 2 * s) * R, 2 * R)], ssem.at[b])

    ldesc(0, 0).start()

    def step(j, carry):
        for b in range(NBUF):
            s = j * NBUF + b

            sb = (b - 1) % NBUF

            @pl.when((s - 1 >= 0) & (s - 1 < nsf))
            def _():
                gdesc(0, s - 1, sb).wait()
                gdesc(1, s - 1, sb).wait()
                sdesc(s - 1, sb).start()

            lb = (b + 1) % NBUF

            @pl.when(s + 1 < nsf)
            def _():
                @pl.when(s + 1 - NBUF >= 0)
                def _():
                    sdesc(s + 1 - NBUF, lb).wait()
                ldesc(s + 1, lb).start()

            @pl.when(s < nsf)
            def _():
                ldesc(s, b).wait()
                gdesc(0, s, b).start(add=True)
                gdesc(1, s, b).start(add=True)
        return carry

    lax.fori_loop(0, (NSTAGE + 1 + NBUF - 1) // NBUF, step, 0)

    for b in range(NBUF):
        sdesc(0, b).wait()

    @pl.when(tail == 1)
    def _():
        k = ni - 1
        half = up_v.at[0, pl.ds(0, R)]
        pltpu.sync_copy(up_hbm.at[pl.ds((c0 + k) * R, R)], half)
        pltpu.async_copy(x_hbm.at[idx_v.at[k]], half, gsem.at[0],
                         add=True).wait()
        pltpu.sync_copy(half, out_hbm.at[pl.ds((c0 + k) * R, R)])


def kernel(x_features, up_x_features, unq_inv):
    idx = unq_inv.astype(jnp.int32)
    pad = (NW * ITERS * R) - M
    idxp = jnp.concatenate([idx, jnp.zeros((pad,), jnp.int32)])
    idxp = idxp.reshape(NW, ITERS, R)
    mesh = plsc.VectorSubcoreMesh(
        core_axis_name="c", subcore_axis_name="s",
        num_cores=NC, num_subcores=NS)
    f = pl.kernel(
        _sc_body,
        out_type=jax.ShapeDtypeStruct((M, C), jnp.float32),
        mesh=mesh,
        scratch_types=[
            pltpu.VMEM((ITERS, R), jnp.int32),
            pltpu.VMEM((NBUF, 2 * R, C), jnp.float32),
            pltpu.SemaphoreType.DMA((NBUF,)),
            pltpu.SemaphoreType.DMA((NBUF,)),
            pltpu.SemaphoreType.DMA((NBUF,)),
        ],
    )
    return f(x_features, up_x_features, idxp)
